# Initial kernel scaffold; baseline (speedup 1.0000x reference)
import jax
import jax.numpy as jnp
from jax.experimental import pallas as pl


def kernel(adj_indices, adj_values, x):
    raise NotImplementedError("write your pallas kernel here")


# speedup vs baseline: 3.3376x; 3.3376x over previous
import functools

import jax
import jax.numpy as jnp
from jax import lax
from jax.experimental import pallas as pl
from jax.experimental.pallas import tpu as pltpu
from jax.experimental.pallas import tpu_sc as plsc

N = 10000
D = 128
NC = 2
NS = 16
NW = NC * NS
K = 128
CH = 80

_mesh = plsc.VectorSubcoreMesh(
    core_axis_name="c", subcore_axis_name="s", num_cores=NC, num_subcores=NS
)


@functools.partial(
    pl.kernel,
    out_type=jax.ShapeDtypeStruct((NC * N, D), jnp.float32),
    mesh=_mesh,
    scratch_types=[
        pltpu.VMEM((CH, K), jnp.int32),
        pltpu.VMEM((CH, K), jnp.int32),
        pltpu.VMEM((CH, K), jnp.float32),
        pltpu.VMEM((K, D), jnp.float32),
        pltpu.VMEM_SHARED((N, D), jnp.float32),
        pltpu.SemaphoreType.DMA,
    ],
)
def _sc_spmm(row_hbm, col_hbm, val_hbm, x_hbm, out_hbm,
             col_v, row_v, val_v, buf, acc, sem):
    c = lax.axis_index("c")
    s = lax.axis_index("s")
    w = s * NC + c

    def _zrow(r, carry):
        for j in range(D // 16):
            buf[r, pl.ds(j * 16, 16)] = jnp.zeros((16,), jnp.float32)
        return carry

    lax.fori_loop(0, K, _zrow, 0)
    for z in range(5):
        idx = s + z * NS

        @pl.when(idx < N // K)
        def _():
            pltpu.sync_copy(buf, acc.at[pl.ds(idx * K, K)])

    @pl.when(s == 0)
    def _():
        pltpu.sync_copy(buf.at[pl.ds(0, N % K)],
                        acc.at[pl.ds((N // K) * K, N % K)])

    plsc.subcore_barrier()

    base = w * CH
    pltpu.sync_copy(col_hbm.at[pl.ds(base, CH)], col_v)
    pltpu.sync_copy(row_hbm.at[pl.ds(base, CH)], row_v)
    pltpu.sync_copy(val_hbm.at[pl.ds(base, CH)], val_v)

    def _chunk(k, carry):
        pltpu.async_copy(x_hbm.at[col_v.at[k]], buf, sem).wait()

        def _edge16(g, inner):
            vv = val_v[k, pl.ds(g * 16, 16)]
            for i in range(16):
                e = g * 16 + i
                v = vv[i]
                for j in range(D // 16):
                    sl = pl.ds(j * 16, 16)
                    buf[e, sl] = buf[e, sl] * v
            return inner

        lax.fori_loop(0, K // 16, _edge16, 0)
        pltpu.sync_copy(buf, acc.at[row_v.at[k]], add=True)
        return carry

    lax.fori_loop(0, CH, _chunk, 0)
    plsc.subcore_barrier()

    for z in range(5):
        idx = s + z * NS

        @pl.when(idx < N // K)
        def _():
            pltpu.sync_copy(acc.at[pl.ds(idx * K, K)],
                            out_hbm.at[pl.ds(c * N + idx * K, K)])

    @pl.when(s == 0)
    def _():
        pltpu.sync_copy(acc.at[pl.ds((N // K) * K, N % K)],
                        out_hbm.at[pl.ds(c * N + (N // K) * K, N % K)])


def _add_body(a_ref, b_ref, o_ref):
    o_ref[...] = a_ref[...] + b_ref[...]


def _combine_partials(partial):
    br = 400
    grid = N // br
    return pl.pallas_call(
        _add_body,
        out_shape=jax.ShapeDtypeStruct((N, D), jnp.float32),
        grid=(grid,),
        in_specs=[
            pl.BlockSpec((br, D), lambda i: (i, 0)),
            pl.BlockSpec((br, D), lambda i: (i + grid, 0)),
        ],
        out_specs=pl.BlockSpec((br, D), lambda i: (i, 0)),
    )(partial, partial)


def kernel(adj_indices, adj_values, x):
    row = adj_indices[0]
    col = adj_indices[1]
    e = adj_values.shape[0]
    ep = NW * CH * K
    pad = ep - e
    row_p = jnp.concatenate([row, jnp.zeros((pad,), jnp.int32)])
    col_p = jnp.concatenate([col, jnp.zeros((pad,), jnp.int32)])
    val_p = jnp.concatenate([adj_values, jnp.zeros((pad,), jnp.float32)])
    row2 = row_p.reshape(NW * CH, K)
    col2 = col_p.reshape(NW * CH, K)
    val2 = val_p.reshape(NW * CH, K)
    partial = _sc_spmm(row2, col2, val2, x)
    return _combine_partials(partial)


# speedup vs baseline: 3.7649x; 1.1280x over previous
import functools

import jax
import jax.numpy as jnp
from jax import lax
from jax.experimental import pallas as pl
from jax.experimental.pallas import tpu as pltpu
from jax.experimental.pallas import tpu_sc as plsc

N = 10000
D = 128
NC = 2
NS = 16
NW = NC * NS
K = 128
CH = 80

_mesh = plsc.VectorSubcoreMesh(
    core_axis_name="c", subcore_axis_name="s", num_cores=NC, num_subcores=NS
)


@functools.partial(
    pl.kernel,
    out_type=jax.ShapeDtypeStruct((NC * N, D), jnp.float32),
    mesh=_mesh,
    scratch_types=[
        pltpu.VMEM((K,), jnp.int32),
        pltpu.VMEM((K,), jnp.int32),
        pltpu.VMEM((K,), jnp.int32),
        pltpu.VMEM((K,), jnp.int32),
        pltpu.VMEM((K,), jnp.float32),
        pltpu.VMEM((K,), jnp.float32),
        pltpu.VMEM((K, D), jnp.float32),
        pltpu.VMEM((K, D), jnp.float32),
        pltpu.VMEM_SHARED((N, D), jnp.float32),
        pltpu.SemaphoreType.DMA,
        pltpu.SemaphoreType.DMA,
        pltpu.SemaphoreType.DMA,
        pltpu.SemaphoreType.DMA,
    ],
)
def _sc_spmm(row_hbm, col_hbm, val_hbm, x_hbm, out_hbm,
             cbuf0, cbuf1, rbuf0, rbuf1, vbuf0, vbuf1, buf0, buf1, acc,
             isem0, isem1, gsem0, gsem1):
    c = lax.axis_index("c")
    s = lax.axis_index("s")
    w = s * NC + c
    ebase = w * CH * K

    cbufs = (cbuf0, cbuf1)
    rbufs = (rbuf0, rbuf1)
    vbufs = (vbuf0, vbuf1)
    bufs = (buf0, buf1)
    isems = (isem0, isem1)
    gsems = (gsem0, gsem1)

    def _zrow(r, carry):
        for j in range(D // 16):
            buf0[r, pl.ds(j * 16, 16)] = jnp.zeros((16,), jnp.float32)
        return carry

    lax.fori_loop(0, K, _zrow, 0)
    for z in range(5):
        idx = s + z * NS

        @pl.when(idx < N // K)
        def _():
            pltpu.sync_copy(buf0, acc.at[pl.ds(idx * K, K)])

    @pl.when(s == 0)
    def _():
        pltpu.sync_copy(buf0.at[pl.ds(0, N % K)],
                        acc.at[pl.ds((N // K) * K, N % K)])

    plsc.subcore_barrier()

    def _idx_dma(k, b):
        off = ebase + k * K
        pltpu.async_copy(col_hbm.at[pl.ds(off, K)], cbufs[b], isems[b])
        pltpu.async_copy(row_hbm.at[pl.ds(off, K)], rbufs[b], isems[b])
        pltpu.async_copy(val_hbm.at[pl.ds(off, K)], vbufs[b], isems[b])

    def _idx_wait(b):
        z = pl.ds(0, K)
        pltpu.make_async_copy(col_hbm.at[z], cbufs[b], isems[b]).wait()
        pltpu.make_async_copy(row_hbm.at[z], rbufs[b], isems[b]).wait()
        pltpu.make_async_copy(val_hbm.at[z], vbufs[b], isems[b]).wait()

    def _gather(b):
        pltpu.async_copy(x_hbm.at[cbufs[b]], bufs[b], gsems[b])

    def _gwait(b):
        pltpu.make_async_copy(x_hbm.at[cbufs[b]], bufs[b], gsems[b]).wait()

    def _process(b):
        def _edge16(g, inner):
            vv = vbufs[b][pl.ds(g * 16, 16)]
            for i in range(16):
                e = g * 16 + i
                v = vv[i]
                for j in range(D // 16):
                    sl = pl.ds(j * 16, 16)
                    bufs[b][e, sl] = bufs[b][e, sl] * v
            return inner

        lax.fori_loop(0, K // 16, _edge16, 0)
        pltpu.sync_copy(bufs[b], acc.at[rbufs[b]], add=True)

    _idx_dma(0, 0)
    _idx_wait(0)
    _gather(0)
    _idx_dma(1, 1)

    def _pair(p, carry):
        for b in range(2):
            k = 2 * p + b
            _gwait(b)
            _idx_wait(1 - b)
            _gather(1 - b)
            _process(b)
            _idx_dma(jnp.minimum(k + 2, CH - 1), b)
        return carry

    lax.fori_loop(0, CH // 2, _pair, 0)
    _gwait(0)
    _idx_wait(1)

    plsc.subcore_barrier()

    for z in range(5):
        idx = s + z * NS

        @pl.when(idx < N // K)
        def _():
            pltpu.sync_copy(acc.at[pl.ds(idx * K, K)],
                            out_hbm.at[pl.ds(c * N + idx * K, K)])

    @pl.when(s == 0)
    def _():
        pltpu.sync_copy(acc.at[pl.ds((N // K) * K, N % K)],
                        out_hbm.at[pl.ds(c * N + (N // K) * K, N % K)])


def _add_body(a_ref, b_ref, o_ref):
    o_ref[...] = a_ref[...] + b_ref[...]


def _combine_partials(partial):
    br = 400
    grid = N // br
    return pl.pallas_call(
        _add_body,
        out_shape=jax.ShapeDtypeStruct((N, D), jnp.float32),
        grid=(grid,),
        in_specs=[
            pl.BlockSpec((br, D), lambda i: (i, 0)),
            pl.BlockSpec((br, D), lambda i: (i + grid, 0)),
        ],
        out_specs=pl.BlockSpec((br, D), lambda i: (i, 0)),
    )(partial, partial)


def kernel(adj_indices, adj_values, x):
    row = adj_indices[0]
    col = adj_indices[1]
    e = adj_values.shape[0]
    ep = NW * CH * K
    pad = ep - e
    row_p = jnp.concatenate([row, jnp.zeros((pad,), jnp.int32)])
    col_p = jnp.concatenate([col, jnp.zeros((pad,), jnp.int32)])
    val_p = jnp.concatenate([adj_values, jnp.zeros((pad,), jnp.float32)])
    partial = _sc_spmm(row_p, col_p, val_p, x)
    return _combine_partials(partial)
